# Initial kernel scaffold; baseline (speedup 1.0000x reference)
import functools

import jax
import jax.numpy as jnp
from jax import lax
from jax.experimental import pallas as pl
from jax.experimental.pallas import tpu as pltpu
from jax.experimental.pallas import tpu_sc as plsc

N = 10000
E = 320000
D = 128
H = 20
C = 2

NP = 10240
W1P = 32
W2P = 16

NC = 2
NS = 16
NW = NC * NS
EPW = E // NW
CH = 80
NSTEP = EPW // CH
NBUF = 5
RPT = NP // NS


def _vmesh():
    return plsc.VectorSubcoreMesh(core_axis_name="c", subcore_axis_name="s")


@jax.jit
def _sc_degree(dst2d):
    @functools.partial(
        pl.kernel,
        out_type=jax.ShapeDtypeStruct((NC, NP, W2P), jnp.float32),
        mesh=_vmesh(),
        scratch_types=[
            pltpu.VMEM((NSTEP, CH), jnp.int32),
            pltpu.VMEM((CH, W2P), jnp.float32),
            pltpu.VMEM((RPT // 16, W2P), jnp.float32),
            pltpu.VMEM_SHARED((NP, W2P), jnp.float32),
            pltpu.SemaphoreType.DMA((NBUF,)),
        ],
    )
    def deg_kernel(dst_hbm, out_hbm, dst_v, ones_v, zbuf_v, acc_sh, sems):
        cid = lax.axis_index("c")
        sid = lax.axis_index("s")
        wid = cid * NS + sid

        pltpu.sync_copy(dst_hbm.at[pl.ds(wid * NSTEP, NSTEP)], dst_v)

        ones16 = jnp.ones((16,), jnp.float32)
        zero16 = jnp.zeros((16,), jnp.float32)

        @pl.loop(0, CH)
        def _(r):
            ones_v[r, pl.ds(0, 16)] = ones16

        @pl.loop(0, RPT // 16)
        def _(r):
            zbuf_v[r, pl.ds(0, 16)] = zero16

        @pl.loop(0, 16)
        def _(j):
            pltpu.sync_copy(
                zbuf_v, acc_sh.at[pl.ds(sid * RPT + j * (RPT // 16), RPT // 16)]
            )

        plsc.subcore_barrier()

        @pl.loop(0, NSTEP, step=NBUF)
        def _(s0):
            descs = []
            for b in range(NBUF):
                descs.append(
                    pltpu.async_copy(
                        ones_v, acc_sh.at[dst_v.at[s0 + b]], sems.at[b], add=True
                    )
                )
            for d in descs:
                d.wait()

        plsc.subcore_barrier()
        pltpu.sync_copy(
            acc_sh.at[pl.ds(sid * RPT, RPT)], out_hbm.at[cid, pl.ds(sid * RPT, RPT)]
        )

    return deg_kernel(dst2d)


def _make_sc_segsum(wd):
    @jax.jit
    def segsum(g, src2d, dst2d):
        @functools.partial(
            pl.kernel,
            out_type=jax.ShapeDtypeStruct((NC, NP, wd), jnp.float32),
            mesh=_vmesh(),
            scratch_types=[
                pltpu.VMEM((NSTEP, CH), jnp.int32),
                pltpu.VMEM((NSTEP, CH), jnp.int32),
                pltpu.VMEM((NBUF, CH, wd), jnp.float32),
                pltpu.VMEM((RPT // 16, wd), jnp.float32),
                pltpu.VMEM_SHARED((NP, wd), jnp.float32),
                pltpu.SemaphoreType.DMA((NBUF,)),
                pltpu.SemaphoreType.DMA((NBUF,)),
            ],
        )
        def seg_kernel(
            g_hbm, src_hbm, dst_hbm, out_hbm,
            src_v, dst_v, rows_v, zbuf_v, acc_sh, gsems, ssems,
        ):
            cid = lax.axis_index("c")
            sid = lax.axis_index("s")
            wid = cid * NS + sid

            pltpu.sync_copy(src_hbm.at[pl.ds(wid * NSTEP, NSTEP)], src_v)
            pltpu.sync_copy(dst_hbm.at[pl.ds(wid * NSTEP, NSTEP)], dst_v)

            zero16 = jnp.zeros((16,), jnp.float32)

            @pl.loop(0, RPT // 16)
            def _(r):
                for c in range(wd // 16):
                    zbuf_v[r, pl.ds(c * 16, 16)] = zero16

            @pl.loop(0, 16)
            def _(j):
                pltpu.sync_copy(
                    zbuf_v,
                    acc_sh.at[pl.ds(sid * RPT + j * (RPT // 16), RPT // 16)],
                )

            plsc.subcore_barrier()

            @pl.loop(0, NSTEP, step=NBUF)
            def _(s0):
                gds = []
                for b in range(NBUF):
                    gds.append(
                        pltpu.async_copy(
                            g_hbm.at[src_v.at[s0 + b]], rows_v.at[b],
                            gsems.at[b],
                        )
                    )
                sds = []
                for b in range(NBUF):
                    gds[b].wait()
                    sds.append(
                        pltpu.async_copy(
                            rows_v.at[b], acc_sh.at[dst_v.at[s0 + b]],
                            ssems.at[b], add=True,
                        )
                    )
                for d in sds:
                    d.wait()

            plsc.subcore_barrier()
            pltpu.sync_copy(
                acc_sh.at[pl.ds(sid * RPT, RPT)],
                out_hbm.at[cid, pl.ds(sid * RPT, RPT)],
            )

        return seg_kernel(g, src2d, dst2d)

    return segsum


_sc_segsum_l1 = _make_sc_segsum(W1P)
_sc_segsum_l2 = _make_sc_segsum(W2P)


def _tc_stage1(x_pad, degp, w1p):
    def body(x_ref, d_ref, w_ref, g_ref):
        deg = d_ref[0, :, 0] + d_ref[1, :, 0] + 1.0
        dis = lax.rsqrt(deg)
        h = jnp.dot(x_ref[...], w_ref[...], precision=lax.Precision.HIGHEST)
        g_ref[...] = h * dis[:, None]

    return pl.pallas_call(
        body,
        out_shape=jax.ShapeDtypeStruct((NP, W1P), jnp.float32),
    )(x_pad, degp, w1p)


def _tc_stage2(s1p, g1, degp, b1p, w2p):
    def body(s_ref, g_ref, d_ref, b_ref, w_ref, o_ref):
        deg = d_ref[0, :, 0] + d_ref[1, :, 0] + 1.0
        dis = lax.rsqrt(deg)[:, None]
        pre = (s_ref[0] + s_ref[1] + g_ref[...]) * dis + b_ref[...]
        act = jnp.where(pre >= 0, pre, 0.01 * pre)
        o_ref[...] = (
            jnp.dot(act, w_ref[...], precision=lax.Precision.HIGHEST) * dis
        )

    return pl.pallas_call(
        body,
        out_shape=jax.ShapeDtypeStruct((NP, W2P), jnp.float32),
    )(s1p, g1, degp, b1p, w2p)


def _tc_stage3(s2p, g2, degp, b2p):
    def body(s_ref, g_ref, d_ref, b_ref, o_ref):
        deg = d_ref[0, :, 0] + d_ref[1, :, 0] + 1.0
        dis = lax.rsqrt(deg)[:, None]
        z = (s_ref[0] + s_ref[1] + g_ref[...]) * dis + b_ref[...]
        z0 = z[:, 0:1]
        z1 = z[:, 1:2]
        m = jnp.maximum(z0, z1)
        lse = m + jnp.log(jnp.exp(z0 - m) + jnp.exp(z1 - m))
        o_ref[...] = jnp.concatenate([z0, z1], axis=1) - lse

    return pl.pallas_call(
        body,
        out_shape=jax.ShapeDtypeStruct((NP, C), jnp.float32),
    )(s2p, g2, degp, b2p)


@jax.jit
def kernel(x, edge_index, W1, b1, W2, b2):
    src2d = edge_index[0].reshape(E // CH, CH)
    dst2d = edge_index[1].reshape(E // CH, CH)

    x_pad = jnp.pad(x, ((0, NP - N), (0, 0)))
    w1p = jnp.pad(W1, ((0, 0), (0, W1P - H)))
    b1p = jnp.pad(b1, (0, W1P - H)).reshape(1, W1P)
    w2p = jnp.pad(W2, ((0, W1P - H), (0, W2P - C)))
    b2p = jnp.pad(b2, (0, W2P - C)).reshape(1, W2P)

    degp = _sc_degree(dst2d)
    g1 = _tc_stage1(x_pad, degp, w1p)
    s1p = _sc_segsum_l1(g1, src2d, dst2d)
    g2 = _tc_stage2(s1p, g1, degp, b1p, w2p)
    s2p = _sc_segsum_l2(g2, src2d, dst2d)
    out = _tc_stage3(s2p, g2, degp, b2p)
    return out[:N]


# speedup vs baseline: 46.9816x; 46.9816x over previous
import functools

import jax
import jax.numpy as jnp
from jax import lax
from jax.experimental import pallas as pl
from jax.experimental.pallas import tpu as pltpu
from jax.experimental.pallas import tpu_sc as plsc

N = 10000
E = 320000
D = 128
H = 20
C = 2

NP = 10240
W1P = 32
W2P = 16

NC = 2
NS = 16
NW = NC * NS
EPW = E // NW
CH = 80
NSTEP = EPW // CH
NBUF = 5
RPT = NP // NS


def _vmesh():
    return plsc.VectorSubcoreMesh(core_axis_name="c", subcore_axis_name="s")


@jax.jit
def _sc_degree(dst2d):
    @functools.partial(
        pl.kernel,
        out_type=jax.ShapeDtypeStruct((NC, NP, W2P), jnp.float32),
        mesh=_vmesh(),
        compiler_params=pltpu.CompilerParams(use_tc_tiling_on_sc=False),
        scratch_types=[
            pltpu.VMEM((NSTEP, CH), jnp.int32),
            pltpu.VMEM((CH, W2P), jnp.float32),
            pltpu.VMEM((RPT // 16, W2P), jnp.float32),
            pltpu.VMEM_SHARED((NP, W2P), jnp.float32),
            pltpu.SemaphoreType.DMA((NBUF,)),
        ],
    )
    def deg_kernel(dst_hbm, out_hbm, dst_v, ones_v, zbuf_v, acc_sh, sems):
        cid = lax.axis_index("c")
        sid = lax.axis_index("s")
        wid = cid * NS + sid

        pltpu.sync_copy(dst_hbm.at[wid], dst_v)

        ones16 = jnp.ones((16,), jnp.float32)
        zero16 = jnp.zeros((16,), jnp.float32)

        @pl.loop(0, CH)
        def _(r):
            ones_v[r, pl.ds(0, 16)] = ones16

        @pl.loop(0, RPT // 16)
        def _(r):
            zbuf_v[r, pl.ds(0, 16)] = zero16

        @pl.loop(0, 16)
        def _(j):
            pltpu.sync_copy(
                zbuf_v, acc_sh.at[pl.ds(sid * RPT + j * (RPT // 16), RPT // 16)]
            )

        plsc.subcore_barrier()

        @pl.loop(0, NSTEP, step=NBUF)
        def _(s0):
            descs = []
            for b in range(NBUF):
                descs.append(
                    pltpu.async_copy(
                        ones_v, acc_sh.at[dst_v.at[s0 + b]], sems.at[b], add=True
                    )
                )
            for d in descs:
                d.wait()

        plsc.subcore_barrier()
        pltpu.sync_copy(
            acc_sh.at[pl.ds(sid * RPT, RPT)], out_hbm.at[cid, pl.ds(sid * RPT, RPT)]
        )

    return deg_kernel(dst2d)


def _make_sc_segsum(wd):
    @jax.jit
    def segsum(g, src2d, dst2d):
        @functools.partial(
            pl.kernel,
            out_type=jax.ShapeDtypeStruct((NC, NP, wd), jnp.float32),
            mesh=_vmesh(),
            compiler_params=pltpu.CompilerParams(use_tc_tiling_on_sc=False),
            scratch_types=[
                pltpu.VMEM((NSTEP, CH), jnp.int32),
                pltpu.VMEM((NSTEP, CH), jnp.int32),
                pltpu.VMEM((NBUF, CH, wd), jnp.float32),
                pltpu.VMEM((RPT // 16, wd), jnp.float32),
                pltpu.VMEM_SHARED((NP, wd), jnp.float32),
                pltpu.SemaphoreType.DMA((NBUF,)),
                pltpu.SemaphoreType.DMA((NBUF,)),
            ],
        )
        def seg_kernel(
            g_hbm, src_hbm, dst_hbm, out_hbm,
            src_v, dst_v, rows_v, zbuf_v, acc_sh, gsems, ssems,
        ):
            cid = lax.axis_index("c")
            sid = lax.axis_index("s")
            wid = cid * NS + sid

            pltpu.sync_copy(src_hbm.at[wid], src_v)
            pltpu.sync_copy(dst_hbm.at[wid], dst_v)

            zero16 = jnp.zeros((16,), jnp.float32)

            @pl.loop(0, RPT // 16)
            def _(r):
                for c in range(wd // 16):
                    zbuf_v[r, pl.ds(c * 16, 16)] = zero16

            @pl.loop(0, 16)
            def _(j):
                pltpu.sync_copy(
                    zbuf_v,
                    acc_sh.at[pl.ds(sid * RPT + j * (RPT // 16), RPT // 16)],
                )

            plsc.subcore_barrier()

            @pl.loop(0, NSTEP, step=NBUF)
            def _(s0):
                gds = []
                for b in range(NBUF):
                    gds.append(
                        pltpu.async_copy(
                            g_hbm.at[src_v.at[s0 + b]], rows_v.at[b],
                            gsems.at[b],
                        )
                    )
                sds = []
                for b in range(NBUF):
                    gds[b].wait()
                    sds.append(
                        pltpu.async_copy(
                            rows_v.at[b], acc_sh.at[dst_v.at[s0 + b]],
                            ssems.at[b], add=True,
                        )
                    )
                for d in sds:
                    d.wait()

            plsc.subcore_barrier()
            pltpu.sync_copy(
                acc_sh.at[pl.ds(sid * RPT, RPT)],
                out_hbm.at[cid, pl.ds(sid * RPT, RPT)],
            )

        return seg_kernel(g, src2d, dst2d)

    return segsum


_sc_segsum_l1 = _make_sc_segsum(W1P)
_sc_segsum_l2 = _make_sc_segsum(W2P)


def _tc_stage1(x_pad, degp, w1p):
    def body(x_ref, d_ref, w_ref, g_ref):
        deg = d_ref[0, :, 0] + d_ref[1, :, 0] + 1.0
        dis = lax.rsqrt(deg)
        h = jnp.dot(x_ref[...], w_ref[...], precision=lax.Precision.HIGHEST)
        g_ref[...] = h * dis[:, None]

    return pl.pallas_call(
        body,
        out_shape=jax.ShapeDtypeStruct((NP, W1P), jnp.float32),
    )(x_pad, degp, w1p)


def _tc_stage2(s1p, g1, degp, b1p, w2p):
    def body(s_ref, g_ref, d_ref, b_ref, w_ref, o_ref):
        deg = d_ref[0, :, 0] + d_ref[1, :, 0] + 1.0
        dis = lax.rsqrt(deg)[:, None]
        pre = (s_ref[0] + s_ref[1] + g_ref[...]) * dis + b_ref[...]
        act = jnp.where(pre >= 0, pre, 0.01 * pre)
        o_ref[...] = (
            jnp.dot(act, w_ref[...], precision=lax.Precision.HIGHEST) * dis
        )

    return pl.pallas_call(
        body,
        out_shape=jax.ShapeDtypeStruct((NP, W2P), jnp.float32),
    )(s1p, g1, degp, b1p, w2p)


def _tc_stage3(s2p, g2, degp, b2p):
    def body(s_ref, g_ref, d_ref, b_ref, o_ref):
        deg = d_ref[0, :, 0] + d_ref[1, :, 0] + 1.0
        dis = lax.rsqrt(deg)[:, None]
        z = (s_ref[0] + s_ref[1] + g_ref[...]) * dis + b_ref[...]
        z0 = z[:, 0:1]
        z1 = z[:, 1:2]
        m = jnp.maximum(z0, z1)
        lse = m + jnp.log(jnp.exp(z0 - m) + jnp.exp(z1 - m))
        o_ref[...] = jnp.concatenate([z0, z1], axis=1) - lse

    return pl.pallas_call(
        body,
        out_shape=jax.ShapeDtypeStruct((NP, C), jnp.float32),
    )(s2p, g2, degp, b2p)


@jax.jit
def kernel(x, edge_index, W1, b1, W2, b2):
    src2d = edge_index[0].reshape(NW, NSTEP, CH)
    dst2d = edge_index[1].reshape(NW, NSTEP, CH)

    x_pad = jnp.pad(x, ((0, NP - N), (0, 0)))
    w1p = jnp.pad(W1, ((0, 0), (0, W1P - H)))
    b1p = jnp.pad(b1, (0, W1P - H)).reshape(1, W1P)
    w2p = jnp.pad(W2, ((0, W1P - H), (0, W2P - C)))
    b2p = jnp.pad(b2, (0, W2P - C)).reshape(1, W2P)

    degp = _sc_degree(dst2d)
    g1 = _tc_stage1(x_pad, degp, w1p)
    s1p = _sc_segsum_l1(g1, src2d, dst2d)
    g2 = _tc_stage2(s1p, g1, degp, b1p, w2p)
    s2p = _sc_segsum_l2(g2, src2d, dst2d)
    out = _tc_stage3(s2p, g2, degp, b2p)
    return out[:N]


# speedup vs baseline: 52.7081x; 1.1219x over previous
import functools

import jax
import jax.numpy as jnp
import numpy as np
from jax import lax
from jax.experimental import pallas as pl
from jax.experimental.pallas import tpu as pltpu
from jax.experimental.pallas import tpu_sc as plsc

N = 10000
E = 320000
D = 128
H = 20
C = 2

NP = 10240
WD = 32
GLR = NP * WD // 128

NC = 2
NS = 16
NW = NC * NS
CH = 128
EP = 327680
EPW = EP // NW
NSTEP = EPW // CH
NBUF = 5
RPT = NP // NS
ORT = GLR // NS


def _vmesh():
    return plsc.VectorSubcoreMesh(core_axis_name="c", subcore_axis_name="s")


_SC_PARAMS = pltpu.CompilerParams(use_tc_tiling_on_sc=False)


def _permute_to_narrow(wide, narrow):
    @pl.loop(0, ORT)
    def _(rr):
        for cc in range(8):
            narrow[rr * 4 + cc // 2, pl.ds((cc % 2) * 16, 16)] = wide[
                rr, pl.ds(cc * 16, 16)
            ]


def _permute_to_wide(narrow, wide):
    @pl.loop(0, ORT)
    def _(rr):
        for cc in range(8):
            wide[rr, pl.ds(cc * 16, 16)] = narrow[
                rr * 4 + cc // 2, pl.ds((cc % 2) * 16, 16)
            ]


@jax.jit
def _sc_degree(dst3d):
    @functools.partial(
        pl.kernel,
        out_type=jax.ShapeDtypeStruct((NC, GLR, 128), jnp.float32),
        mesh=_vmesh(),
        compiler_params=_SC_PARAMS,
        scratch_types=[
            pltpu.VMEM((NSTEP, CH), jnp.int32),
            pltpu.VMEM((CH, WD), jnp.float32),
            pltpu.VMEM((RPT, WD), jnp.float32),
            pltpu.VMEM((ORT, 128), jnp.float32),
            pltpu.VMEM_SHARED((NP, WD), jnp.float32),
            pltpu.SemaphoreType.DMA((NBUF,)),
        ],
    )
    def deg_kernel(dst_hbm, out_hbm, dst_v, ones_v, qbuf, pbuf, acc_sh, sems):
        cid = lax.axis_index("c")
        sid = lax.axis_index("s")
        wid = cid * NS + sid

        pltpu.sync_copy(dst_hbm.at[wid], dst_v)

        ones16 = jnp.ones((16,), jnp.float32)
        zero16 = jnp.zeros((16,), jnp.float32)

        @pl.loop(0, CH)
        def _(r):
            for c in range(WD // 16):
                ones_v[r, pl.ds(c * 16, 16)] = ones16

        @pl.loop(0, RPT)
        def _(r):
            for c in range(WD // 16):
                qbuf[r, pl.ds(c * 16, 16)] = zero16

        pltpu.sync_copy(qbuf, acc_sh.at[pl.ds(sid * RPT, RPT)])
        plsc.subcore_barrier()

        @pl.loop(0, NSTEP, step=NBUF)
        def _(s0):
            descs = []
            for b in range(NBUF):
                descs.append(
                    pltpu.async_copy(
                        ones_v, acc_sh.at[dst_v.at[s0 + b]], sems.at[b], add=True
                    )
                )
            for d in descs:
                d.wait()

        plsc.subcore_barrier()
        pltpu.sync_copy(acc_sh.at[pl.ds(sid * RPT, RPT)], qbuf)
        _permute_to_wide(qbuf, pbuf)
        pltpu.sync_copy(pbuf, out_hbm.at[cid, pl.ds(sid * ORT, ORT)])

    return deg_kernel(dst3d)


@jax.jit
def _sc_segsum(gl, src3d, dst3d):
    @functools.partial(
        pl.kernel,
        out_type=jax.ShapeDtypeStruct((NC, GLR, 128), jnp.float32),
        mesh=_vmesh(),
        compiler_params=_SC_PARAMS,
        scratch_types=[
            pltpu.VMEM((NSTEP, CH), jnp.int32),
            pltpu.VMEM((NSTEP, CH), jnp.int32),
            pltpu.VMEM((NBUF, CH, WD), jnp.float32),
            pltpu.VMEM((RPT, WD), jnp.float32),
            pltpu.VMEM((ORT, 128), jnp.float32),
            pltpu.VMEM_SHARED((NP, WD), jnp.float32),
            pltpu.VMEM_SHARED((NP, WD), jnp.float32),
            pltpu.SemaphoreType.DMA((NBUF,)),
            pltpu.SemaphoreType.DMA((NBUF,)),
        ],
    )
    def seg_kernel(
        g_hbm, src_hbm, dst_hbm, out_hbm,
        src_v, dst_v, rows_v, qbuf, pbuf, tbl_sh, acc_sh, gsems, ssems,
    ):
        cid = lax.axis_index("c")
        sid = lax.axis_index("s")
        wid = cid * NS + sid

        pltpu.sync_copy(src_hbm.at[wid], src_v)
        pltpu.sync_copy(dst_hbm.at[wid], dst_v)

        pltpu.sync_copy(g_hbm.at[pl.ds(sid * ORT, ORT)], pbuf)
        _permute_to_narrow(pbuf, qbuf)
        pltpu.sync_copy(qbuf, tbl_sh.at[pl.ds(sid * RPT, RPT)])

        zero16 = jnp.zeros((16,), jnp.float32)

        @pl.loop(0, RPT)
        def _(r):
            for c in range(WD // 16):
                qbuf[r, pl.ds(c * 16, 16)] = zero16

        pltpu.sync_copy(qbuf, acc_sh.at[pl.ds(sid * RPT, RPT)])
        plsc.subcore_barrier()

        @pl.loop(0, NSTEP, step=NBUF)
        def _(s0):
            gds = []
            for b in range(NBUF):
                gds.append(
                    pltpu.async_copy(
                        tbl_sh.at[src_v.at[s0 + b]], rows_v.at[b],
                        gsems.at[b],
                    )
                )
            sds = []
            for b in range(NBUF):
                gds[b].wait()
                sds.append(
                    pltpu.async_copy(
                        rows_v.at[b], acc_sh.at[dst_v.at[s0 + b]],
                        ssems.at[b], add=True,
                    )
                )
            for d in sds:
                d.wait()

        plsc.subcore_barrier()
        pltpu.sync_copy(acc_sh.at[pl.ds(sid * RPT, RPT)], qbuf)
        _permute_to_wide(qbuf, pbuf)
        pltpu.sync_copy(pbuf, out_hbm.at[cid, pl.ds(sid * ORT, ORT)])

    return seg_kernel(gl, src3d, dst3d)


def _tc_h1(x4, w1bd):
    def body(x_ref, w_ref, h_ref):
        h_ref[...] = jnp.dot(
            x_ref[...], w_ref[...], precision=lax.Precision.HIGHEST
        )

    return pl.pallas_call(
        body, out_shape=jax.ShapeDtypeStruct((GLR, 128), jnp.float32)
    )(x4, w1bd)


def _tc_g1(h1l, degp):
    def body(h_ref, d_ref, g_ref):
        dis = lax.rsqrt(d_ref[0] + d_ref[1] + 1.0)
        g_ref[...] = h_ref[...] * dis

    return pl.pallas_call(
        body, out_shape=jax.ShapeDtypeStruct((GLR, 128), jnp.float32)
    )(h1l, degp)


def _tc_stage2(s1p, g1l, degp, b1bc, w2bd):
    def body(s_ref, g_ref, d_ref, b_ref, w_ref, o_ref):
        dis = lax.rsqrt(d_ref[0] + d_ref[1] + 1.0)
        pre = (s_ref[0] + s_ref[1] + g_ref[...]) * dis + b_ref[...]
        act = jnp.where(pre >= 0, pre, 0.01 * pre)
        o_ref[...] = (
            jnp.dot(act, w_ref[...], precision=lax.Precision.HIGHEST) * dis
        )

    return pl.pallas_call(
        body, out_shape=jax.ShapeDtypeStruct((GLR, 128), jnp.float32)
    )(s1p, g1l, degp, b1bc, w2bd)


def _tc_stage3(s2p, g2l, degp, b2bc, sel_a, sel_b):
    def body(s_ref, g_ref, d_ref, b_ref, sa_ref, sb_ref, o_ref):
        dis = lax.rsqrt(d_ref[0] + d_ref[1] + 1.0)
        z = (s_ref[0] + s_ref[1] + g_ref[...]) * dis + b_ref[...]
        za = jnp.dot(z, sa_ref[...], precision=lax.Precision.HIGHEST)
        zb = jnp.dot(z, sb_ref[...], precision=lax.Precision.HIGHEST)
        m = jnp.maximum(za, zb)
        lse = m + jnp.log(jnp.exp(za - m) + jnp.exp(zb - m))
        o_ref[...] = za - lse

    return pl.pallas_call(
        body, out_shape=jax.ShapeDtypeStruct((GLR, 8), jnp.float32)
    )(s2p, g2l, degp, b2bc, sel_a, sel_b)


def _block_diag(w, nblk, bin_, bout):
    out = jnp.zeros((nblk * bin_, nblk * bout), w.dtype)
    for i in range(nblk):
        out = out.at[
            i * bin_ : i * bin_ + w.shape[0], i * bout : i * bout + w.shape[1]
        ].set(w)
    return out


_SEL_A = np.zeros((128, 8), np.float32)
_SEL_B = np.zeros((128, 8), np.float32)
for _j in range(4):
    _SEL_A[32 * _j + 0, 2 * _j + 0] = 1.0
    _SEL_A[32 * _j + 1, 2 * _j + 1] = 1.0
    _SEL_B[32 * _j + 1, 2 * _j + 0] = 1.0
    _SEL_B[32 * _j + 0, 2 * _j + 1] = 1.0


@jax.jit
def kernel(x, edge_index, W1, b1, W2, b2):
    ep = jnp.pad(edge_index, ((0, 0), (0, EP - E)), constant_values=NP - 1)
    src3d = ep[0].reshape(NW, NSTEP, CH)
    dst3d = ep[1].reshape(NW, NSTEP, CH)

    x4 = jnp.pad(x, ((0, NP - N), (0, 0))).reshape(GLR, 4 * D)
    w1p = jnp.pad(W1, ((0, 0), (0, WD - H)))
    w1bd = _block_diag(w1p, 4, D, WD)
    w2p = jnp.pad(W2, ((0, WD - H), (0, WD - C)))
    w2bd = _block_diag(w2p, 4, WD, WD)
    b1bc = jnp.tile(jnp.pad(b1, (0, WD - H)), 4).reshape(1, 128)
    b2bc = jnp.tile(jnp.pad(b2, (0, WD - C)), 4).reshape(1, 128)
    sel_a = jnp.asarray(_SEL_A)
    sel_b = jnp.asarray(_SEL_B)

    h1l = _tc_h1(x4, w1bd)
    degp = _sc_degree(dst3d)
    g1l = _tc_g1(h1l, degp)
    s1p = _sc_segsum(g1l, src3d, dst3d)
    g2l = _tc_stage2(s1p, g1l, degp, b1bc, w2bd)
    s2p = _sc_segsum(g2l, src3d, dst3d)
    out8 = _tc_stage3(s2p, g2l, degp, b2bc, sel_a, sel_b)
    return out8.reshape(NP, C)[:N]
